# baseline (speedup 1.0000x reference)
import functools

import jax
import jax.numpy as jnp
from jax import lax
from jax.experimental import pallas as pl
from jax.experimental.pallas import tpu as pltpu
from jax.experimental.pallas import tpu_sc as plsc

_TC_BLOCK = 4096
_NCOLS = 64
_LANES = 16


def _topk_compact_body(x_ref, w_ref, wout_ref, iout_ref):
    x = x_ref[...]
    w = w_ref[...]
    b = x.shape[0]
    logits = jax.lax.dot_general(
        w, x, (((1,), (1,)), ((), ())), preferred_element_type=jnp.float32
    )
    m = jnp.max(logits, axis=0, keepdims=True)
    e = jnp.exp(logits - m)
    s = jnp.sum(e, axis=0, keepdims=True)
    p = e / s

    iota = jax.lax.broadcasted_iota(jnp.int32, p.shape, 0)
    big = jnp.int32(_NCOLS)

    v1 = jnp.max(p, axis=0, keepdims=True)
    i1 = jnp.min(jnp.where(p == v1, iota, big), axis=0, keepdims=True)
    p_m = jnp.where(iota == i1, -jnp.inf, p)
    v2 = jnp.max(p_m, axis=0, keepdims=True)
    i2 = jnp.min(jnp.where(p_m == v2, iota, big), axis=0, keepdims=True)

    t = jnp.exp(v2 - v1)
    denom = 1.0 + t
    w1 = 1.0 / denom
    w2 = t / denom

    zf = jnp.zeros((6, b), jnp.float32)
    zi = jnp.zeros((6, b), jnp.int32)
    wout_ref[...] = jnp.concatenate([w1, w2, zf], axis=0)
    iout_ref[...] = jnp.concatenate([i1, i2, zi], axis=0)


@jax.jit
def _topk_compact(x, W):
    n, dim = x.shape
    nmoe = W.shape[0]
    grid = (n // _TC_BLOCK,)
    return pl.pallas_call(
        _topk_compact_body,
        grid=grid,
        in_specs=[
            pl.BlockSpec((_TC_BLOCK, dim), lambda i: (i, 0)),
            pl.BlockSpec((nmoe, dim), lambda i: (0, 0)),
        ],
        out_specs=[
            pl.BlockSpec((8, _TC_BLOCK), lambda i: (0, i)),
            pl.BlockSpec((8, _TC_BLOCK), lambda i: (0, i)),
        ],
        out_shape=[
            jax.ShapeDtypeStruct((8, n), jnp.float32),
            jax.ShapeDtypeStruct((8, n), jnp.int32),
        ],
    )(x, W)


def _make_scatter_kernel(n_tokens):
    info = plsc.get_sparse_core_info()
    nc, ns = info.num_cores, info.num_subcores
    nw = nc * ns
    n_per = n_tokens // nw
    mesh = plsc.VectorSubcoreMesh(core_axis_name="c", subcore_axis_name="s")

    @functools.partial(
        pl.kernel,
        out_type=jax.ShapeDtypeStruct((n_tokens * _NCOLS,), jnp.float32),
        mesh=mesh,
        compiler_params=pltpu.CompilerParams(needs_layout_passes=False),
        scratch_types=[
            pltpu.VMEM((n_per,), jnp.float32),
            pltpu.VMEM((n_per,), jnp.float32),
            pltpu.VMEM((n_per,), jnp.int32),
            pltpu.VMEM((n_per,), jnp.int32),
            pltpu.VMEM((n_per * _NCOLS,), jnp.float32),
        ],
    )
    def scatter_kernel(w_hbm, i_hbm, out_hbm, w1_v, w2_v, i1_v, i2_v, buf_v):
        wid = lax.axis_index("s") * nc + lax.axis_index("c")
        base = wid * n_per
        pltpu.sync_copy(w_hbm.at[0, pl.ds(base, n_per)], w1_v)
        pltpu.sync_copy(w_hbm.at[1, pl.ds(base, n_per)], w2_v)
        pltpu.sync_copy(i_hbm.at[0, pl.ds(base, n_per)], i1_v)
        pltpu.sync_copy(i_hbm.at[1, pl.ds(base, n_per)], i2_v)

        zeros16 = jnp.zeros((_LANES,), jnp.float32)

        def zero_body(r, carry):
            buf_v[pl.ds(r * _LANES, _LANES)] = zeros16
            return carry

        lax.fori_loop(0, n_per * _NCOLS // _LANES, zero_body, 0, unroll=8)

        lane_iota = lax.iota(jnp.int32, _LANES)

        def scat_body(j, carry):
            off = j * _LANES
            rowbase = (lane_iota + off) * _NCOLS
            cols1 = i1_v[pl.ds(off, _LANES)]
            vals1 = w1_v[pl.ds(off, _LANES)]
            plsc.store_scatter(buf_v, [rowbase + cols1], vals1)
            cols2 = i2_v[pl.ds(off, _LANES)]
            vals2 = w2_v[pl.ds(off, _LANES)]
            plsc.store_scatter(buf_v, [rowbase + cols2], vals2)
            return carry

        lax.fori_loop(0, n_per // _LANES, scat_body, 0, unroll=4)

        pltpu.sync_copy(buf_v, out_hbm.at[pl.ds(base * _NCOLS, n_per * _NCOLS)])

    return scatter_kernel


@jax.jit
def _gating(x, W):
    n = x.shape[0]
    nmoe = W.shape[0]
    wc, ic = _topk_compact(x, W)
    flat = _make_scatter_kernel(n)(wc, ic)
    return jnp.reshape(flat, (n, nmoe))


def kernel(x, W, topk):
    del topk
    return _gating(x, W)


# speedup vs baseline: 4.2578x; 4.2578x over previous
import functools

import jax
import jax.numpy as jnp
from jax.experimental import pallas as pl
from jax.experimental.pallas import tpu as pltpu

_BLOCK = 2048


def _gating_t_body(xt_ref, w_ref, out_ref):
    xt = xt_ref[...]
    w = w_ref[...]
    logits = jax.lax.dot_general(
        w, xt, (((1,), (0,)), ((), ())), preferred_element_type=jnp.float32
    )
    m = jnp.max(logits, axis=0, keepdims=True)
    e = jnp.exp(logits - m)
    s = jnp.sum(e, axis=0, keepdims=True)
    p = e / s

    nmoe = p.shape[0]
    iota = jax.lax.broadcasted_iota(jnp.int32, p.shape, 0)
    big = jnp.int32(nmoe)

    v1 = jnp.max(p, axis=0, keepdims=True)
    i1 = jnp.min(jnp.where(p == v1, iota, big), axis=0, keepdims=True)
    p_m = jnp.where(iota == i1, -jnp.inf, p)
    v2 = jnp.max(p_m, axis=0, keepdims=True)
    i2 = jnp.min(jnp.where(p_m == v2, iota, big), axis=0, keepdims=True)

    t = jnp.exp(v2 - v1)
    denom = 1.0 + t
    w1 = 1.0 / denom
    w2 = t / denom

    out_ref[...] = jnp.where(
        iota == i1, w1, jnp.where(iota == i2, w2, jnp.float32(0.0))
    )


@jax.jit
def _gating(x, W):
    n, dim = x.shape
    nmoe = W.shape[0]
    xt = jnp.transpose(x)
    grid = (n // _BLOCK,)
    out_t = pl.pallas_call(
        _gating_t_body,
        grid=grid,
        in_specs=[
            pl.BlockSpec((dim, _BLOCK), lambda i: (0, i)),
            pl.BlockSpec((nmoe, dim), lambda i: (0, 0)),
        ],
        out_specs=pl.BlockSpec((nmoe, _BLOCK), lambda i: (0, i)),
        out_shape=jax.ShapeDtypeStruct((nmoe, n), jnp.float32),
    )(xt, W)
    return jnp.transpose(out_t)


def kernel(x, W, topk):
    del topk
    return _gating(x, W)
